# baseline (speedup 1.0000x reference)
import functools

import jax
import jax.numpy as jnp
from jax import lax
from jax.experimental import pallas as pl
from jax.experimental.pallas import tpu as pltpu
from jax.experimental.pallas import tpu_sc as plsc

N_NODES_C = 10000
D_C = 128
NC = 2
NS = 16
NW = NC * NS
CHUNK = 128
ROWS_PAD = 10112
ROWS_PER_TILE = ROWS_PAD // NS
ZROWS = 128
DEG_W = 128

_mesh = plsc.VectorSubcoreMesh(core_axis_name="c", subcore_axis_name="s")


_TILE_PIECES = [(o, min(ZROWS, ROWS_PER_TILE - o)) for o in range(0, ROWS_PER_TILE, ZROWS)]


def _zero_tile_slice(acc_sp, zeros_hbm, stage_v, s):
  pltpu.sync_copy(zeros_hbm, stage_v)
  for o, ln in _TILE_PIECES:
    pltpu.sync_copy(stage_v.at[pl.ds(0, ln)],
                    acc_sp.at[pl.ds(s * ROWS_PER_TILE + o, ln)])


def _write_tile_slice(acc_sp, out_hbm, stage_v, c, s):
  for o, ln in _TILE_PIECES:
    sl = pl.ds(s * ROWS_PER_TILE + o, ln)
    pltpu.sync_copy(acc_sp.at[sl], stage_v.at[pl.ds(0, ln)])
    pltpu.sync_copy(stage_v.at[pl.ds(0, ln)], out_hbm.at[c, sl])


def _deg_kernel(dst_hbm, ones_hbm, zeros_hbm, out_hbm, deg_sp, idx_v, ones_v, zb_v):
  c = lax.axis_index("c")
  s = lax.axis_index("s")
  nch = dst_hbm.shape[2]
  _zero_tile_slice(deg_sp, zeros_hbm, zb_v, s)
  pltpu.sync_copy(ones_hbm, ones_v)
  pltpu.sync_copy(dst_hbm.at[c, s], idx_v)
  plsc.subcore_barrier()

  def body(j, carry):
    pltpu.sync_copy(ones_v, deg_sp.at[idx_v.at[j]], add=True)
    return carry

  lax.fori_loop(0, nch, body, 0)
  plsc.subcore_barrier()
  _write_tile_slice(deg_sp, out_hbm, zb_v, c, s)


def _agg_kernel(z_hbm, src_hbm, dst_hbm, zeros_hbm, out_hbm,
                acc_sp, src_v, dst_v, rows_v):
  c = lax.axis_index("c")
  s = lax.axis_index("s")
  nch = src_hbm.shape[2]
  _zero_tile_slice(acc_sp, zeros_hbm, rows_v, s)
  pltpu.sync_copy(src_hbm.at[c, s], src_v)
  pltpu.sync_copy(dst_hbm.at[c, s], dst_v)
  plsc.subcore_barrier()

  def body(j, carry):
    pltpu.sync_copy(z_hbm.at[src_v.at[j]], rows_v)
    pltpu.sync_copy(rows_v, acc_sp.at[dst_v.at[j]], add=True)
    return carry

  lax.fori_loop(0, nch, body, 0)
  plsc.subcore_barrier()
  _write_tile_slice(acc_sp, out_hbm, rows_v, c, s)


def _make_sc_calls(n_chunks):
  deg_call = pl.kernel(
      _deg_kernel,
      out_type=jax.ShapeDtypeStruct((NC, ROWS_PAD, DEG_W), jnp.float32),
      mesh=_mesh,
      scratch_types=[
          pltpu.VMEM_SHARED((ROWS_PAD, DEG_W), jnp.float32),
          pltpu.VMEM((n_chunks, CHUNK), jnp.int32),
          pltpu.VMEM((CHUNK, DEG_W), jnp.float32),
          pltpu.VMEM((ZROWS, DEG_W), jnp.float32),
      ],
  )
  agg_call = pl.kernel(
      _agg_kernel,
      out_type=jax.ShapeDtypeStruct((NC, ROWS_PAD, D_C), jnp.float32),
      mesh=_mesh,
      scratch_types=[
          pltpu.VMEM_SHARED((ROWS_PAD, D_C), jnp.float32),
          pltpu.VMEM((n_chunks, CHUNK), jnp.int32),
          pltpu.VMEM((n_chunks, CHUNK), jnp.int32),
          pltpu.VMEM((CHUNK, D_C), jnp.float32),
      ],
  )
  return deg_call, agg_call


_BM = 1000


def _tc_a_body(x_ref, w_ref, d0_ref, d1_ref, z_ref, dis_ref):
  deg = d0_ref[...] + d1_ref[...] + 1.0
  dis = jax.lax.rsqrt(deg)
  dis_ref[...] = dis
  z_ref[...] = jnp.dot(x_ref[...], w_ref[...],
                       preferred_element_type=jnp.float32) * dis


def _tc_b_body(a0_ref, a1_ref, z1_ref, dis_ref, b_ref, w_ref, z2_ref):
  dis = dis_ref[...]
  h = (a0_ref[...] + a1_ref[...] + z1_ref[...]) * dis + b_ref[...]
  h = jnp.maximum(h, 0.0)
  z2_ref[...] = jnp.dot(h, w_ref[...], preferred_element_type=jnp.float32) * dis


def _tc_c_body(a0_ref, a1_ref, z2_ref, dis_ref, b_ref, out_ref):
  out_ref[...] = (a0_ref[...] + a1_ref[...] + z2_ref[...]) * dis_ref[...] + b_ref[...]


def _row_spec(width):
  return pl.BlockSpec((_BM, width), lambda i: (i, 0))


def _full_spec(rows, cols):
  return pl.BlockSpec((rows, cols), lambda i: (0, 0))


def _tc_stage_a(x, w1, d0, d1):
  n = x.shape[0]
  grid = (n // _BM,)
  return pl.pallas_call(
      _tc_a_body,
      grid=grid,
      in_specs=[_row_spec(D_C), _full_spec(D_C, D_C), _row_spec(1), _row_spec(1)],
      out_specs=[_row_spec(D_C), _row_spec(1)],
      out_shape=[jax.ShapeDtypeStruct((n, D_C), jnp.float32),
                 jax.ShapeDtypeStruct((n, 1), jnp.float32)],
  )(x, w1, d0, d1)


def _tc_stage_b(a0, a1, z1, dis, b1, w2):
  n = z1.shape[0]
  grid = (n // _BM,)
  return pl.pallas_call(
      _tc_b_body,
      grid=grid,
      in_specs=[_row_spec(D_C), _row_spec(D_C), _row_spec(D_C), _row_spec(1),
                _full_spec(1, D_C), _full_spec(D_C, D_C)],
      out_specs=_row_spec(D_C),
      out_shape=jax.ShapeDtypeStruct((n, D_C), jnp.float32),
  )(a0, a1, z1, dis, b1, w2)


def _tc_stage_c(a0, a1, z2, dis, b2):
  n = z2.shape[0]
  grid = (n // _BM,)
  return pl.pallas_call(
      _tc_c_body,
      grid=grid,
      in_specs=[_row_spec(D_C), _row_spec(D_C), _row_spec(D_C), _row_spec(1),
                _full_spec(1, D_C)],
      out_specs=_row_spec(D_C),
      out_shape=jax.ShapeDtypeStruct((n, D_C), jnp.float32),
  )(a0, a1, z2, dis, b2)


def kernel(x, edge_index, W1, b1, W2, b2):
  n = x.shape[0]
  e = edge_index.shape[1]
  n_chunks = -(-e // (NW * CHUNK))
  n_chunks += n_chunks % 2
  e_pad = NW * n_chunks * CHUNK
  src = edge_index[0].astype(jnp.int32)
  dst = edge_index[1].astype(jnp.int32)
  pad = e_pad - e
  src_p = jnp.concatenate([src, jnp.zeros((pad,), jnp.int32)]
                          ).reshape(NC, NS, n_chunks, CHUNK)
  dst_p = jnp.concatenate([dst, jnp.full((pad,), n, jnp.int32)]
                          ).reshape(NC, NS, n_chunks, CHUNK)

  ones_rows = jnp.ones((CHUNK, DEG_W), jnp.float32)
  zeros_rows = jnp.zeros((ZROWS, D_C), jnp.float32)

  deg_call, agg_call = _make_sc_calls(n_chunks)

  deg_parts = deg_call(dst_p, ones_rows, zeros_rows)
  d0 = lax.slice(deg_parts, (0, 0, 0), (1, n, 1)).reshape(n, 1)
  d1 = lax.slice(deg_parts, (1, 0, 0), (2, n, 1)).reshape(n, 1)

  z1, dis = _tc_stage_a(x, W1, d0, d1)

  acc1 = agg_call(z1, src_p, dst_p, zeros_rows)
  a0 = lax.slice(acc1, (0, 0, 0), (1, n, D_C)).reshape(n, D_C)
  a1 = lax.slice(acc1, (1, 0, 0), (2, n, D_C)).reshape(n, D_C)

  b1r = b1.reshape(1, D_C)
  z2 = _tc_stage_b(a0, a1, z1, dis, b1r, W2)

  acc2 = agg_call(z2, src_p, dst_p, zeros_rows)
  c0 = lax.slice(acc2, (0, 0, 0), (1, n, D_C)).reshape(n, D_C)
  c1 = lax.slice(acc2, (1, 0, 0), (2, n, D_C)).reshape(n, D_C)

  b2r = b2.reshape(1, D_C)
  return _tc_stage_c(c0, c1, z2, dis, b2r)


# speedup vs baseline: 2.3033x; 2.3033x over previous
import functools

import jax
import jax.numpy as jnp
from jax import lax
from jax.experimental import pallas as pl
from jax.experimental.pallas import tpu as pltpu
from jax.experimental.pallas import tpu_sc as plsc

N_NODES_C = 10000
D_C = 128
NC = 2
NS = 16
NW = NC * NS
CHUNK = 128
ROWS_PAD = 10112
ROWS_PER_TILE = ROWS_PAD // NS
ZROWS = 128
DEG_W = 128

_mesh = plsc.VectorSubcoreMesh(core_axis_name="c", subcore_axis_name="s")


_TILE_PIECES = [(o, min(ZROWS, ROWS_PER_TILE - o)) for o in range(0, ROWS_PER_TILE, ZROWS)]


def _zero_tile_slice(acc_sp, zeros_hbm, stage_v, s):
  pltpu.sync_copy(zeros_hbm, stage_v)
  for o, ln in _TILE_PIECES:
    pltpu.sync_copy(stage_v.at[pl.ds(0, ln)],
                    acc_sp.at[pl.ds(s * ROWS_PER_TILE + o, ln)])


def _write_tile_slice(acc_sp, out_hbm, stage_v, c, s):
  for o, ln in _TILE_PIECES:
    sl = pl.ds(s * ROWS_PER_TILE + o, ln)
    pltpu.sync_copy(acc_sp.at[sl], stage_v.at[pl.ds(0, ln)])
    pltpu.sync_copy(stage_v.at[pl.ds(0, ln)], out_hbm.at[c, sl])


def _deg_kernel(dst_hbm, ones_hbm, zeros_hbm, out_hbm, deg_sp, idx_v, ones_v, zb_v):
  c = lax.axis_index("c")
  s = lax.axis_index("s")
  nch = dst_hbm.shape[2]
  _zero_tile_slice(deg_sp, zeros_hbm, zb_v, s)
  pltpu.sync_copy(ones_hbm, ones_v)
  pltpu.sync_copy(dst_hbm.at[c, s], idx_v)
  plsc.subcore_barrier()

  def body(j, carry):
    pltpu.sync_copy(ones_v, deg_sp.at[idx_v.at[j]], add=True)
    return carry

  lax.fori_loop(0, nch, body, 0)
  plsc.subcore_barrier()
  _write_tile_slice(deg_sp, out_hbm, zb_v, c, s)


def _agg_kernel(z_hbm, src_hbm, dst_hbm, zeros_hbm, out_hbm,
                acc_sp, src_v, dst_v, rows_v):
  c = lax.axis_index("c")
  s = lax.axis_index("s")
  nch = src_hbm.shape[2]
  _zero_tile_slice(acc_sp, zeros_hbm, rows_v, s)
  pltpu.sync_copy(src_hbm.at[c, s], src_v)
  pltpu.sync_copy(dst_hbm.at[c, s], dst_v)
  plsc.subcore_barrier()

  def body(j, carry):
    pltpu.sync_copy(z_hbm.at[src_v.at[j]], rows_v)
    pltpu.sync_copy(rows_v, acc_sp.at[dst_v.at[j]], add=True)
    return carry

  lax.fori_loop(0, nch, body, 0)
  plsc.subcore_barrier()
  _write_tile_slice(acc_sp, out_hbm, rows_v, c, s)


def _make_sc_calls(n_chunks):
  deg_call = pl.kernel(
      _deg_kernel,
      out_type=jax.ShapeDtypeStruct((NC, ROWS_PAD, DEG_W), jnp.float32),
      mesh=_mesh,
      scratch_types=[
          pltpu.VMEM_SHARED((ROWS_PAD, DEG_W), jnp.float32),
          pltpu.VMEM((n_chunks, CHUNK), jnp.int32),
          pltpu.VMEM((CHUNK, DEG_W), jnp.float32),
          pltpu.VMEM((ZROWS, DEG_W), jnp.float32),
      ],
  )
  agg_call = pl.kernel(
      _agg_kernel,
      out_type=jax.ShapeDtypeStruct((NC, ROWS_PAD, D_C), jnp.float32),
      mesh=_mesh,
      scratch_types=[
          pltpu.VMEM_SHARED((ROWS_PAD, D_C), jnp.float32),
          pltpu.VMEM((n_chunks, CHUNK), jnp.int32),
          pltpu.VMEM((n_chunks, CHUNK), jnp.int32),
          pltpu.VMEM((CHUNK, D_C), jnp.float32),
      ],
  )
  return deg_call, agg_call


_BM = 1000


def _tc_a_body(x_ref, w_ref, d0_ref, d1_ref, z_ref, dis_ref):
  deg = d0_ref[...] + d1_ref[...] + 1.0
  dis = jax.lax.rsqrt(deg)
  dis_ref[...] = dis
  z_ref[...] = jnp.dot(x_ref[...], w_ref[...],
                       preferred_element_type=jnp.float32) * dis


def _tc_b_body(a0_ref, a1_ref, z1_ref, dis_ref, b_ref, w_ref, z2_ref):
  dis = dis_ref[...]
  h = (a0_ref[...] + a1_ref[...] + z1_ref[...]) * dis + b_ref[...]
  h = jnp.maximum(h, 0.0)
  z2_ref[...] = jnp.dot(h, w_ref[...], preferred_element_type=jnp.float32) * dis


def _tc_c_body(a0_ref, a1_ref, z2_ref, dis_ref, b_ref, out_ref):
  out_ref[...] = (a0_ref[...] + a1_ref[...] + z2_ref[...]) * dis_ref[...] + b_ref[...]


def _row_spec(width):
  return pl.BlockSpec((_BM, width), lambda i: (i, 0))


def _full_spec(rows, cols):
  return pl.BlockSpec((rows, cols), lambda i: (0, 0))


def _tc_stage_a(x, w1, d0, d1):
  n = x.shape[0]
  grid = (n // _BM,)
  return pl.pallas_call(
      _tc_a_body,
      grid=grid,
      in_specs=[_row_spec(D_C), _full_spec(D_C, D_C), _row_spec(1), _row_spec(1)],
      out_specs=[_row_spec(D_C), _row_spec(1)],
      out_shape=[jax.ShapeDtypeStruct((n, D_C), jnp.float32),
                 jax.ShapeDtypeStruct((n, 1), jnp.float32)],
  )(x, w1, d0, d1)


def _tc_stage_b(a0, a1, z1, dis, b1, w2):
  n = z1.shape[0]
  grid = (n // _BM,)
  return pl.pallas_call(
      _tc_b_body,
      grid=grid,
      in_specs=[_row_spec(D_C), _row_spec(D_C), _row_spec(D_C), _row_spec(1),
                _full_spec(1, D_C), _full_spec(D_C, D_C)],
      out_specs=_row_spec(D_C),
      out_shape=jax.ShapeDtypeStruct((n, D_C), jnp.float32),
  )(a0, a1, z1, dis, b1, w2)


def _tc_stage_c(a0, a1, z2, dis, b2):
  n = z2.shape[0]
  grid = (n // _BM,)
  return pl.pallas_call(
      _tc_c_body,
      grid=grid,
      in_specs=[_row_spec(D_C), _row_spec(D_C), _row_spec(D_C), _row_spec(1),
                _full_spec(1, D_C)],
      out_specs=_row_spec(D_C),
      out_shape=jax.ShapeDtypeStruct((n, D_C), jnp.float32),
  )(a0, a1, z2, dis, b2)


def kernel(x, edge_index, W1, b1, W2, b2):
  n = x.shape[0]
  e = edge_index.shape[1]
  n_chunks = -(-e // (NW * CHUNK))
  n_chunks += n_chunks % 2
  e_pad = NW * n_chunks * CHUNK
  src = edge_index[0].astype(jnp.int32)
  dst = edge_index[1].astype(jnp.int32)
  pad = e_pad - e
  pad_ar = jnp.arange(pad, dtype=jnp.int32)
  src_p = jnp.concatenate([src, pad_ar % n]).reshape(NC, NS, n_chunks, CHUNK)
  dst_p = jnp.concatenate([dst, n + pad_ar % (ROWS_PAD - n)]
                          ).reshape(NC, NS, n_chunks, CHUNK)

  ones_rows = jnp.ones((CHUNK, DEG_W), jnp.float32)
  zeros_rows = jnp.zeros((ZROWS, D_C), jnp.float32)

  deg_call, agg_call = _make_sc_calls(n_chunks)

  deg_parts = deg_call(dst_p, ones_rows, zeros_rows)
  d0 = lax.slice(deg_parts, (0, 0, 0), (1, n, 1)).reshape(n, 1)
  d1 = lax.slice(deg_parts, (1, 0, 0), (2, n, 1)).reshape(n, 1)

  z1, dis = _tc_stage_a(x, W1, d0, d1)

  acc1 = agg_call(z1, src_p, dst_p, zeros_rows)
  a0 = lax.slice(acc1, (0, 0, 0), (1, n, D_C)).reshape(n, D_C)
  a1 = lax.slice(acc1, (1, 0, 0), (2, n, D_C)).reshape(n, D_C)

  b1r = b1.reshape(1, D_C)
  z2 = _tc_stage_b(a0, a1, z1, dis, b1r, W2)

  acc2 = agg_call(z2, src_p, dst_p, zeros_rows)
  c0 = lax.slice(acc2, (0, 0, 0), (1, n, D_C)).reshape(n, D_C)
  c1 = lax.slice(acc2, (1, 0, 0), (2, n, D_C)).reshape(n, D_C)

  b2r = b2.reshape(1, D_C)
  return _tc_stage_c(c0, c1, z2, dis, b2r)


# speedup vs baseline: 3.1487x; 1.3670x over previous
import functools

import jax
import jax.numpy as jnp
from jax import lax
from jax.experimental import pallas as pl
from jax.experimental.pallas import tpu as pltpu
from jax.experimental.pallas import tpu_sc as plsc

N_NODES_C = 10000
D_C = 128
NC = 2
NS = 16
NW = NC * NS
CHUNK = 128
ROWS_PAD = 10112
ROWS_PER_TILE = ROWS_PAD // NS
ZROWS = 128
DEG_W = 128

_mesh = plsc.VectorSubcoreMesh(core_axis_name="c", subcore_axis_name="s")


_TILE_PIECES = [(o, min(ZROWS, ROWS_PER_TILE - o)) for o in range(0, ROWS_PER_TILE, ZROWS)]


def _zero_tile_slice(acc_sp, zeros_hbm, stage_v, s):
  pltpu.sync_copy(zeros_hbm, stage_v)
  for o, ln in _TILE_PIECES:
    pltpu.sync_copy(stage_v.at[pl.ds(0, ln)],
                    acc_sp.at[pl.ds(s * ROWS_PER_TILE + o, ln)])


def _write_tile_slice(acc_sp, out_hbm, stage_v, c, s):
  for o, ln in _TILE_PIECES:
    sl = pl.ds(s * ROWS_PER_TILE + o, ln)
    pltpu.sync_copy(acc_sp.at[sl], stage_v.at[pl.ds(0, ln)])
    pltpu.sync_copy(stage_v.at[pl.ds(0, ln)], out_hbm.at[c, sl])


def _deg_kernel(dst_hbm, ones_hbm, zeros_hbm, out_hbm, deg_sp, idx_v, ones_v, zb_v):
  c = lax.axis_index("c")
  s = lax.axis_index("s")
  nch = dst_hbm.shape[2]
  _zero_tile_slice(deg_sp, zeros_hbm, zb_v, s)
  pltpu.sync_copy(ones_hbm, ones_v)
  pltpu.sync_copy(dst_hbm.at[c, s], idx_v)
  plsc.subcore_barrier()

  def body(j, carry):
    pltpu.sync_copy(ones_v, deg_sp.at[idx_v.at[j]], add=True)
    return carry

  lax.fori_loop(0, nch, body, 0)
  plsc.subcore_barrier()
  _write_tile_slice(deg_sp, out_hbm, zb_v, c, s)


def _agg_kernel(z_hbm, pk_hbm, zeros_hbm, out_hbm, acc_sp, pk_v,
                rows0, rows1, sidx0, sidx1, didx0, didx1, semg0, semg1):
  c = lax.axis_index("c")
  s = lax.axis_index("s")
  nch = pk_hbm.shape[2]
  nh = nch // 2
  _zero_tile_slice(acc_sp, zeros_hbm, rows0, s)
  pltpu.sync_copy(pk_hbm.at[c, s], pk_v)
  plsc.subcore_barrier()

  def unpack(j, sidx, didx):
    for k in range(CHUNK // 16):
      v = pk_v[j, pl.ds(k * 16, 16)]
      sidx[0, pl.ds(k * 16, 16)] = v & 16383
      didx[0, pl.ds(k * 16, 16)] = v >> 14

  def gather(j, sidx, rows, semg):
    unpack_and_go = pltpu.async_copy(z_hbm.at[sidx.at[0]], rows, semg)
    return unpack_and_go

  def finish(sidx, didx, rows, semg):
    pltpu.make_async_copy(z_hbm.at[sidx.at[0]], rows, semg).wait()
    pltpu.sync_copy(rows, acc_sp.at[didx.at[0]], add=True)

  unpack(0, sidx0, didx0)
  gather(0, sidx0, rows0, semg0)

  def body(i, carry):
    j0 = 2 * i
    unpack(j0 + 1, sidx1, didx1)
    gather(j0 + 1, sidx1, rows1, semg1)
    finish(sidx0, didx0, rows0, semg0)

    @pl.when(i + 1 < nh)
    def _():
      unpack(j0 + 2, sidx0, didx0)
      gather(j0 + 2, sidx0, rows0, semg0)

    finish(sidx1, didx1, rows1, semg1)
    return carry

  lax.fori_loop(0, nh, body, 0)
  plsc.subcore_barrier()
  _write_tile_slice(acc_sp, out_hbm, rows0, c, s)


def _make_sc_calls(n_chunks):
  deg_call = pl.kernel(
      _deg_kernel,
      out_type=jax.ShapeDtypeStruct((NC, ROWS_PAD, DEG_W), jnp.float32),
      mesh=_mesh,
      scratch_types=[
          pltpu.VMEM_SHARED((ROWS_PAD, DEG_W), jnp.float32),
          pltpu.VMEM((n_chunks, CHUNK), jnp.int32),
          pltpu.VMEM((CHUNK, DEG_W), jnp.float32),
          pltpu.VMEM((ZROWS, DEG_W), jnp.float32),
      ],
  )
  agg_call = pl.kernel(
      _agg_kernel,
      out_type=jax.ShapeDtypeStruct((NC, ROWS_PAD, D_C), jnp.float32),
      mesh=_mesh,
      scratch_types=[
          pltpu.VMEM_SHARED((ROWS_PAD, D_C), jnp.float32),
          pltpu.VMEM((n_chunks, CHUNK), jnp.int32),
          pltpu.VMEM((CHUNK, D_C), jnp.float32),
          pltpu.VMEM((CHUNK, D_C), jnp.float32),
          pltpu.VMEM((1, CHUNK), jnp.int32),
          pltpu.VMEM((1, CHUNK), jnp.int32),
          pltpu.VMEM((1, CHUNK), jnp.int32),
          pltpu.VMEM((1, CHUNK), jnp.int32),
          pltpu.SemaphoreType.DMA,
          pltpu.SemaphoreType.DMA,
      ],
  )
  return deg_call, agg_call


_BM = 1000


def _tc_a_body(x_ref, w_ref, d0_ref, d1_ref, z_ref, dis_ref):
  deg = d0_ref[...] + d1_ref[...] + 1.0
  dis = jax.lax.rsqrt(deg)
  dis_ref[...] = dis
  z_ref[...] = jnp.dot(x_ref[...], w_ref[...],
                       preferred_element_type=jnp.float32) * dis


def _tc_b_body(a0_ref, a1_ref, z1_ref, dis_ref, b_ref, w_ref, z2_ref):
  dis = dis_ref[...]
  h = (a0_ref[...] + a1_ref[...] + z1_ref[...]) * dis + b_ref[...]
  h = jnp.maximum(h, 0.0)
  z2_ref[...] = jnp.dot(h, w_ref[...], preferred_element_type=jnp.float32) * dis


def _tc_c_body(a0_ref, a1_ref, z2_ref, dis_ref, b_ref, out_ref):
  out_ref[...] = (a0_ref[...] + a1_ref[...] + z2_ref[...]) * dis_ref[...] + b_ref[...]


def _row_spec(width):
  return pl.BlockSpec((_BM, width), lambda i: (i, 0))


def _full_spec(rows, cols):
  return pl.BlockSpec((rows, cols), lambda i: (0, 0))


def _tc_stage_a(x, w1, d0, d1):
  n = x.shape[0]
  grid = (n // _BM,)
  return pl.pallas_call(
      _tc_a_body,
      grid=grid,
      in_specs=[_row_spec(D_C), _full_spec(D_C, D_C), _row_spec(1), _row_spec(1)],
      out_specs=[_row_spec(D_C), _row_spec(1)],
      out_shape=[jax.ShapeDtypeStruct((n, D_C), jnp.float32),
                 jax.ShapeDtypeStruct((n, 1), jnp.float32)],
  )(x, w1, d0, d1)


def _tc_stage_b(a0, a1, z1, dis, b1, w2):
  n = z1.shape[0]
  grid = (n // _BM,)
  return pl.pallas_call(
      _tc_b_body,
      grid=grid,
      in_specs=[_row_spec(D_C), _row_spec(D_C), _row_spec(D_C), _row_spec(1),
                _full_spec(1, D_C), _full_spec(D_C, D_C)],
      out_specs=_row_spec(D_C),
      out_shape=jax.ShapeDtypeStruct((n, D_C), jnp.float32),
  )(a0, a1, z1, dis, b1, w2)


def _tc_stage_c(a0, a1, z2, dis, b2):
  n = z2.shape[0]
  grid = (n // _BM,)
  return pl.pallas_call(
      _tc_c_body,
      grid=grid,
      in_specs=[_row_spec(D_C), _row_spec(D_C), _row_spec(D_C), _row_spec(1),
                _full_spec(1, D_C)],
      out_specs=_row_spec(D_C),
      out_shape=jax.ShapeDtypeStruct((n, D_C), jnp.float32),
  )(a0, a1, z2, dis, b2)


def kernel(x, edge_index, W1, b1, W2, b2):
  n = x.shape[0]
  e = edge_index.shape[1]
  n_chunks = -(-e // (NW * CHUNK))
  n_chunks += n_chunks % 2
  e_pad = NW * n_chunks * CHUNK
  src = edge_index[0].astype(jnp.int32)
  dst = edge_index[1].astype(jnp.int32)
  pad = e_pad - e
  pad_ar = jnp.arange(pad, dtype=jnp.int32)
  src_a = jnp.concatenate([src, pad_ar % n])
  dst_a = jnp.concatenate([dst, n + pad_ar % (ROWS_PAD - n)])
  dst_p = dst_a.reshape(NC, NS, n_chunks, CHUNK)
  pk_p = ((dst_a << 14) | src_a).reshape(NC, NS, n_chunks, CHUNK)

  ones_rows = jnp.ones((CHUNK, DEG_W), jnp.float32)
  zeros_rows = jnp.zeros((ZROWS, D_C), jnp.float32)

  deg_call, agg_call = _make_sc_calls(n_chunks)

  deg_parts = deg_call(dst_p, ones_rows, zeros_rows)
  d0 = lax.slice(deg_parts, (0, 0, 0), (1, n, 1)).reshape(n, 1)
  d1 = lax.slice(deg_parts, (1, 0, 0), (2, n, 1)).reshape(n, 1)

  z1, dis = _tc_stage_a(x, W1, d0, d1)

  acc1 = agg_call(z1, pk_p, zeros_rows)
  a0 = lax.slice(acc1, (0, 0, 0), (1, n, D_C)).reshape(n, D_C)
  a1 = lax.slice(acc1, (1, 0, 0), (2, n, D_C)).reshape(n, D_C)

  b1r = b1.reshape(1, D_C)
  z2 = _tc_stage_b(a0, a1, z1, dis, b1r, W2)

  acc2 = agg_call(z2, pk_p, zeros_rows)
  c0 = lax.slice(acc2, (0, 0, 0), (1, n, D_C)).reshape(n, D_C)
  c1 = lax.slice(acc2, (1, 0, 0), (2, n, D_C)).reshape(n, D_C)

  b2r = b2.reshape(1, D_C)
  return _tc_stage_c(c0, c1, z2, dis, b2r)


# speedup vs baseline: 3.8298x; 1.2163x over previous
import functools

import jax
import jax.numpy as jnp
from jax import lax
from jax.experimental import pallas as pl
from jax.experimental.pallas import tpu as pltpu
from jax.experimental.pallas import tpu_sc as plsc

N_NODES_C = 10000
D_C = 128
NC = 2
NS = 16
NW = NC * NS
CHUNK = 128
ROWS_PAD = 10112
ROWS_PER_TILE = ROWS_PAD // NS
ZROWS = 128

_mesh = plsc.VectorSubcoreMesh(core_axis_name="c", subcore_axis_name="s")


_TILE_PIECES = [(o, min(ZROWS, ROWS_PER_TILE - o)) for o in range(0, ROWS_PER_TILE, ZROWS)]


def _zero_tile_slice(acc_sp, zeros_hbm, stage_v, s):
  pltpu.sync_copy(zeros_hbm, stage_v)
  for o, ln in _TILE_PIECES:
    pltpu.sync_copy(stage_v.at[pl.ds(0, ln)],
                    acc_sp.at[pl.ds(s * ROWS_PER_TILE + o, ln)])


def _write_tile_slice(acc_sp, out_hbm, stage_v, c, s):
  for o, ln in _TILE_PIECES:
    sl = pl.ds(s * ROWS_PER_TILE + o, ln)
    pltpu.sync_copy(acc_sp.at[sl], stage_v.at[pl.ds(0, ln)])
    pltpu.sync_copy(stage_v.at[pl.ds(0, ln)], out_hbm.at[c, sl])


def _deg_kernel(dst_hbm, out_hbm, hist_v, idx_v):
  c = lax.axis_index("c")
  s = lax.axis_index("s")
  nch = dst_hbm.shape[2]
  zero16 = jnp.zeros((16,), jnp.float32)

  def zbody(i, carry):
    hist_v[pl.ds(i * 16, 16)] = zero16
    return carry

  lax.fori_loop(0, ROWS_PAD // 16, zbody, 0)
  pltpu.sync_copy(dst_hbm.at[c, s], idx_v)
  ones16 = jnp.ones((16,), jnp.float32)

  def body(j, carry):
    for k in range(CHUNK // 16):
      idx16 = idx_v[j, pl.ds(k * 16, 16)]
      plsc.addupdate_scatter(hist_v, [idx16], ones16)
    return carry

  lax.fori_loop(0, nch, body, 0)
  pltpu.sync_copy(hist_v, out_hbm.at[c, s])


def _agg_kernel(z_hbm, pk_hbm, zeros_hbm, out_hbm, acc_sp, pk_v,
                rows0, rows1, sidx0, sidx1, didx0, didx1, semg0, semg1):
  c = lax.axis_index("c")
  s = lax.axis_index("s")
  nch = pk_hbm.shape[2] // CHUNK
  nh = nch // 2
  _zero_tile_slice(acc_sp, zeros_hbm, rows0, s)
  pltpu.sync_copy(pk_hbm.at[c, s], pk_v)
  plsc.subcore_barrier()

  def unpack(j, sidx, didx):
    for k in range(CHUNK // 16):
      v = pk_v[pl.ds(j * CHUNK + k * 16, 16)]
      sidx[pl.ds(k * 16, 16)] = v & 16383
      didx[pl.ds(k * 16, 16)] = v >> 14

  def gather(j, sidx, rows, semg):
    return pltpu.async_copy(z_hbm.at[sidx], rows, semg)

  def finish(sidx, didx, rows, semg):
    pltpu.make_async_copy(z_hbm.at[sidx], rows, semg).wait()
    pltpu.sync_copy(rows, acc_sp.at[didx], add=True)

  unpack(0, sidx0, didx0)
  gather(0, sidx0, rows0, semg0)

  def body(i, carry):
    j0 = 2 * i
    unpack(j0 + 1, sidx1, didx1)
    gather(j0 + 1, sidx1, rows1, semg1)
    finish(sidx0, didx0, rows0, semg0)

    @pl.when(i + 1 < nh)
    def _():
      unpack(j0 + 2, sidx0, didx0)
      gather(j0 + 2, sidx0, rows0, semg0)

    finish(sidx1, didx1, rows1, semg1)
    return carry

  lax.fori_loop(0, nh, body, 0)
  plsc.subcore_barrier()
  _write_tile_slice(acc_sp, out_hbm, rows0, c, s)


def _make_sc_calls(n_chunks):
  deg_call = pl.kernel(
      _deg_kernel,
      out_type=jax.ShapeDtypeStruct((NC, NS, ROWS_PAD), jnp.float32),
      mesh=_mesh,
      scratch_types=[
          pltpu.VMEM((ROWS_PAD,), jnp.float32),
          pltpu.VMEM((n_chunks, CHUNK), jnp.int32),
      ],
      compiler_params=pltpu.CompilerParams(needs_layout_passes=False),
  )
  agg_call = pl.kernel(
      _agg_kernel,
      out_type=jax.ShapeDtypeStruct((NC, ROWS_PAD, D_C), jnp.float32),
      mesh=_mesh,
      scratch_types=[
          pltpu.VMEM_SHARED((ROWS_PAD, D_C), jnp.float32),
          pltpu.VMEM((n_chunks * CHUNK,), jnp.int32),
          pltpu.VMEM((CHUNK, D_C), jnp.float32),
          pltpu.VMEM((CHUNK, D_C), jnp.float32),
          pltpu.VMEM((CHUNK,), jnp.int32),
          pltpu.VMEM((CHUNK,), jnp.int32),
          pltpu.VMEM((CHUNK,), jnp.int32),
          pltpu.VMEM((CHUNK,), jnp.int32),
          pltpu.SemaphoreType.DMA,
          pltpu.SemaphoreType.DMA,
      ],
      compiler_params=pltpu.CompilerParams(needs_layout_passes=False),
  )
  return deg_call, agg_call


_BM = 1000


def _tc_a_body(x_ref, w_ref, dh_ref, z_ref, dis_ref):
  deg = dh_ref[...] + 1.0
  dis = jax.lax.rsqrt(deg)
  dis_ref[...] = dis
  z_ref[...] = jnp.dot(x_ref[...], w_ref[...],
                       preferred_element_type=jnp.float32) * dis


def _tc_b_body(a0_ref, a1_ref, z1_ref, dis_ref, b_ref, w_ref, z2_ref):
  dis = dis_ref[...]
  h = (a0_ref[...] + a1_ref[...] + z1_ref[...]) * dis + b_ref[...]
  h = jnp.maximum(h, 0.0)
  z2_ref[...] = jnp.dot(h, w_ref[...], preferred_element_type=jnp.float32) * dis


def _tc_c_body(a0_ref, a1_ref, z2_ref, dis_ref, b_ref, out_ref):
  out_ref[...] = (a0_ref[...] + a1_ref[...] + z2_ref[...]) * dis_ref[...] + b_ref[...]


def _row_spec(width):
  return pl.BlockSpec((_BM, width), lambda i: (i, 0))


def _full_spec(rows, cols):
  return pl.BlockSpec((rows, cols), lambda i: (0, 0))


def _tc_stage_a(x, w1, degh):
  n = x.shape[0]
  grid = (n // _BM,)
  return pl.pallas_call(
      _tc_a_body,
      grid=grid,
      in_specs=[_row_spec(D_C), _full_spec(D_C, D_C), _row_spec(1)],
      out_specs=[_row_spec(D_C), _row_spec(1)],
      out_shape=[jax.ShapeDtypeStruct((n, D_C), jnp.float32),
                 jax.ShapeDtypeStruct((n, 1), jnp.float32)],
  )(x, w1, degh)


def _tc_stage_b(a0, a1, z1, dis, b1, w2):
  n = z1.shape[0]
  grid = (n // _BM,)
  return pl.pallas_call(
      _tc_b_body,
      grid=grid,
      in_specs=[_row_spec(D_C), _row_spec(D_C), _row_spec(D_C), _row_spec(1),
                _full_spec(1, D_C), _full_spec(D_C, D_C)],
      out_specs=_row_spec(D_C),
      out_shape=jax.ShapeDtypeStruct((n, D_C), jnp.float32),
  )(a0, a1, z1, dis, b1, w2)


def _tc_stage_c(a0, a1, z2, dis, b2):
  n = z2.shape[0]
  grid = (n // _BM,)
  return pl.pallas_call(
      _tc_c_body,
      grid=grid,
      in_specs=[_row_spec(D_C), _row_spec(D_C), _row_spec(D_C), _row_spec(1),
                _full_spec(1, D_C)],
      out_specs=_row_spec(D_C),
      out_shape=jax.ShapeDtypeStruct((n, D_C), jnp.float32),
  )(a0, a1, z2, dis, b2)


def kernel(x, edge_index, W1, b1, W2, b2):
  n = x.shape[0]
  e = edge_index.shape[1]
  n_chunks = -(-e // (NW * CHUNK))
  n_chunks += n_chunks % 2
  e_pad = NW * n_chunks * CHUNK
  src = edge_index[0].astype(jnp.int32)
  dst = edge_index[1].astype(jnp.int32)
  pad = e_pad - e
  pad_ar = jnp.arange(pad, dtype=jnp.int32)
  src_a = jnp.concatenate([src, pad_ar % n])
  dst_a = jnp.concatenate([dst, n + pad_ar % (ROWS_PAD - n)])
  dst_p = dst_a.reshape(NC, NS, n_chunks, CHUNK)
  pk_p = ((dst_a << 14) | src_a).reshape(NC, NS, n_chunks * CHUNK)


  zeros_rows = jnp.zeros((ZROWS, D_C), jnp.float32)

  deg_call, agg_call = _make_sc_calls(n_chunks)

  deg_parts = deg_call(dst_p)
  degh = deg_parts.reshape(NW, ROWS_PAD).sum(axis=0)[:n].reshape(n, 1)

  z1, dis = _tc_stage_a(x, W1, degh)

  acc1 = agg_call(z1, pk_p, zeros_rows)
  a0 = lax.slice(acc1, (0, 0, 0), (1, n, D_C)).reshape(n, D_C)
  a1 = lax.slice(acc1, (1, 0, 0), (2, n, D_C)).reshape(n, D_C)

  b1r = b1.reshape(1, D_C)
  z2 = _tc_stage_b(a0, a1, z1, dis, b1r, W2)

  acc2 = agg_call(z2, pk_p, zeros_rows)
  c0 = lax.slice(acc2, (0, 0, 0), (1, n, D_C)).reshape(n, D_C)
  c1 = lax.slice(acc2, (1, 0, 0), (2, n, D_C)).reshape(n, D_C)

  b2r = b2.reshape(1, D_C)
  return _tc_stage_c(c0, c1, z2, dis, b2r)


# speedup vs baseline: 3.9944x; 1.0430x over previous
import functools

import jax
import jax.numpy as jnp
from jax import lax
from jax.experimental import pallas as pl
from jax.experimental.pallas import tpu as pltpu
from jax.experimental.pallas import tpu_sc as plsc

N_NODES_C = 10000
D_C = 128
NC = 2
NS = 16
NW = NC * NS
CHUNK = 128
ROWS_PAD = 10112
ROWS_PER_TILE = ROWS_PAD // NS
ZROWS = 128

_mesh = plsc.VectorSubcoreMesh(core_axis_name="c", subcore_axis_name="s")


_TILE_PIECES = [(o, min(ZROWS, ROWS_PER_TILE - o)) for o in range(0, ROWS_PER_TILE, ZROWS)]


def _zero_tile_slice(acc_sp, zeros_hbm, stage_v, s):
  pltpu.sync_copy(zeros_hbm, stage_v)
  for o, ln in _TILE_PIECES:
    pltpu.sync_copy(stage_v.at[pl.ds(0, ln)],
                    acc_sp.at[pl.ds(s * ROWS_PER_TILE + o, ln)])


def _write_tile_slice(acc_sp, out_hbm, stage_v, c, s):
  for o, ln in _TILE_PIECES:
    sl = pl.ds(s * ROWS_PER_TILE + o, ln)
    pltpu.sync_copy(acc_sp.at[sl], stage_v.at[pl.ds(0, ln)])
    pltpu.sync_copy(stage_v.at[pl.ds(0, ln)], out_hbm.at[c, sl])


def _deg_kernel(dst_hbm, out_hbm, hist_v, idx_v):
  c = lax.axis_index("c")
  s = lax.axis_index("s")
  nch = dst_hbm.shape[2]
  zero16 = jnp.zeros((16,), jnp.float32)

  def zbody(i, carry):
    hist_v[pl.ds(i * 16, 16)] = zero16
    return carry

  lax.fori_loop(0, ROWS_PAD // 16, zbody, 0)
  pltpu.sync_copy(dst_hbm.at[c, s], idx_v)
  ones16 = jnp.ones((16,), jnp.float32)

  def body(j, carry):
    for k in range(CHUNK // 16):
      idx16 = idx_v[j, pl.ds(k * 16, 16)]
      plsc.addupdate_scatter(hist_v, [idx16], ones16)
    return carry

  lax.fori_loop(0, nch, body, 0)
  pltpu.sync_copy(hist_v, out_hbm.at[c, s])


def _agg_kernel(z_hbm, pk_hbm, zeros_hbm, out_hbm, acc_sp, pk_v,
                rows0, rows1, sidx0, sidx1, didx0, didx1, semg0, semg1):
  c = lax.axis_index("c")
  s = lax.axis_index("s")
  nch = pk_hbm.shape[2] // CHUNK
  nh = nch // 2
  _zero_tile_slice(acc_sp, zeros_hbm, rows0, s)
  pltpu.sync_copy(pk_hbm.at[c, s], pk_v)
  plsc.subcore_barrier()

  def unpack(j, sidx, didx):
    for k in range(CHUNK // 16):
      v = pk_v[pl.ds(j * CHUNK + k * 16, 16)]
      sidx[pl.ds(k * 16, 16)] = v & 16383
      didx[pl.ds(k * 16, 16)] = v >> 14

  def gather(j, sidx, rows, semg):
    return pltpu.async_copy(z_hbm.at[sidx], rows, semg)

  def finish(sidx, didx, rows, semg):
    pltpu.make_async_copy(z_hbm.at[sidx], rows, semg).wait()
    pltpu.sync_copy(rows, acc_sp.at[didx], add=True)

  unpack(0, sidx0, didx0)
  gather(0, sidx0, rows0, semg0)

  def body(i, carry):
    j0 = 2 * i
    unpack(j0 + 1, sidx1, didx1)
    gather(j0 + 1, sidx1, rows1, semg1)
    finish(sidx0, didx0, rows0, semg0)

    @pl.when(i + 1 < nh)
    def _():
      unpack(j0 + 2, sidx0, didx0)
      gather(j0 + 2, sidx0, rows0, semg0)

    finish(sidx1, didx1, rows1, semg1)
    return carry

  lax.fori_loop(0, nh, body, 0)
  plsc.subcore_barrier()
  _write_tile_slice(acc_sp, out_hbm, rows0, c, s)


def _make_sc_calls(n_chunks):
  deg_call = pl.kernel(
      _deg_kernel,
      out_type=jax.ShapeDtypeStruct((NC, NS, ROWS_PAD), jnp.float32),
      mesh=_mesh,
      scratch_types=[
          pltpu.VMEM((ROWS_PAD,), jnp.float32),
          pltpu.VMEM((n_chunks, CHUNK), jnp.int32),
      ],
      compiler_params=pltpu.CompilerParams(needs_layout_passes=False),
  )
  agg_call = pl.kernel(
      _agg_kernel,
      out_type=jax.ShapeDtypeStruct((NC, ROWS_PAD, D_C), jnp.float32),
      mesh=_mesh,
      scratch_types=[
          pltpu.VMEM_SHARED((ROWS_PAD, D_C), jnp.float32),
          pltpu.VMEM((n_chunks * CHUNK,), jnp.int32),
          pltpu.VMEM((CHUNK, D_C), jnp.float32),
          pltpu.VMEM((CHUNK, D_C), jnp.float32),
          pltpu.VMEM((CHUNK,), jnp.int32),
          pltpu.VMEM((CHUNK,), jnp.int32),
          pltpu.VMEM((CHUNK,), jnp.int32),
          pltpu.VMEM((CHUNK,), jnp.int32),
          pltpu.SemaphoreType.DMA,
          pltpu.SemaphoreType.DMA,
      ],
      compiler_params=pltpu.CompilerParams(needs_layout_passes=False),
  )
  return deg_call, agg_call


_BM = 1000


def _tc_a_body(x_ref, w_ref, dh_ref, z_ref, dis_ref):
  deg = dh_ref[...] + 1.0
  dis = jax.lax.rsqrt(deg)
  dis_ref[...] = dis
  z_ref[...] = jnp.dot(x_ref[...], w_ref[...],
                       preferred_element_type=jnp.float32) * dis


def _tc_b_body(a0_ref, a1_ref, z1_ref, dis_ref, b_ref, w_ref, z2_ref):
  dis = dis_ref[...]
  h = (a0_ref[0] + a1_ref[0] + z1_ref[...]) * dis + b_ref[...]
  h = jnp.maximum(h, 0.0)
  z2_ref[...] = jnp.dot(h, w_ref[...], preferred_element_type=jnp.float32) * dis


def _tc_c_body(a0_ref, a1_ref, z2_ref, dis_ref, b_ref, out_ref):
  out_ref[...] = (a0_ref[0] + a1_ref[0] + z2_ref[...]) * dis_ref[...] + b_ref[...]


def _row_spec(width):
  return pl.BlockSpec((_BM, width), lambda i: (i, 0))


def _full_spec(rows, cols):
  return pl.BlockSpec((rows, cols), lambda i: (0, 0))


def _tc_stage_a(x, w1, degh):
  n = x.shape[0]
  grid = (n // _BM,)
  return pl.pallas_call(
      _tc_a_body,
      grid=grid,
      in_specs=[_row_spec(D_C), _full_spec(D_C, D_C), _row_spec(1)],
      out_specs=[_row_spec(D_C), _row_spec(1)],
      out_shape=[jax.ShapeDtypeStruct((n, D_C), jnp.float32),
                 jax.ShapeDtypeStruct((n, 1), jnp.float32)],
  )(x, w1, degh)


def _acc_spec(core):
  return pl.BlockSpec((1, _BM, D_C), lambda i, core=core: (core, i, 0))


def _tc_stage_b(acc, z1, dis, b1, w2):
  n = z1.shape[0]
  grid = (n // _BM,)
  return pl.pallas_call(
      _tc_b_body,
      grid=grid,
      in_specs=[_acc_spec(0), _acc_spec(1), _row_spec(D_C), _row_spec(1),
                _full_spec(1, D_C), _full_spec(D_C, D_C)],
      out_specs=_row_spec(D_C),
      out_shape=jax.ShapeDtypeStruct((n, D_C), jnp.float32),
  )(acc, acc, z1, dis, b1, w2)


def _tc_stage_c(acc, z2, dis, b2):
  n = z2.shape[0]
  grid = (n // _BM,)
  return pl.pallas_call(
      _tc_c_body,
      grid=grid,
      in_specs=[_acc_spec(0), _acc_spec(1), _row_spec(D_C), _row_spec(1),
                _full_spec(1, D_C)],
      out_specs=_row_spec(D_C),
      out_shape=jax.ShapeDtypeStruct((n, D_C), jnp.float32),
  )(acc, acc, z2, dis, b2)


def kernel(x, edge_index, W1, b1, W2, b2):
  n = x.shape[0]
  e = edge_index.shape[1]
  n_chunks = -(-e // (NW * CHUNK))
  n_chunks += n_chunks % 2
  e_pad = NW * n_chunks * CHUNK
  src = edge_index[0].astype(jnp.int32)
  dst = edge_index[1].astype(jnp.int32)
  pad = e_pad - e
  pad_ar = jnp.arange(pad, dtype=jnp.int32)
  src_a = jnp.concatenate([src, pad_ar % n])
  dst_a = jnp.concatenate([dst, n + pad_ar % (ROWS_PAD - n)])
  dst_p = dst_a.reshape(NC, NS, n_chunks, CHUNK)
  pk_p = ((dst_a << 14) | src_a).reshape(NC, NS, n_chunks * CHUNK)


  zeros_rows = jnp.zeros((ZROWS, D_C), jnp.float32)

  deg_call, agg_call = _make_sc_calls(n_chunks)

  deg_parts = deg_call(dst_p)
  degh = deg_parts.reshape(NW, ROWS_PAD).sum(axis=0)[:n].reshape(n, 1)

  z1, dis = _tc_stage_a(x, W1, degh)

  acc1 = agg_call(z1, pk_p, zeros_rows)
  z2 = _tc_stage_b(acc1, z1, dis, b1.reshape(1, D_C), W2)

  acc2 = agg_call(z2, pk_p, zeros_rows)
  return _tc_stage_c(acc2, z2, dis, b2.reshape(1, D_C))
